# baseline (speedup 1.0000x reference)
import functools
import math

import jax
import jax.numpy as jnp
from jax import lax
from jax.experimental import pallas as pl
from jax.experimental.pallas import tpu as pltpu
from jax.experimental.pallas import tpu_sc as plsc

_B, _CTX, _D, _V = 1024, 20, 64, 100000
_NC, _NS, _L = 2, 16, 16
_NW = _NC * _NS
_BPW = _B // _NW
_IPW = _BPW * _CTX
_DCHUNKS = _D // _L

@functools.cache
def _gather_sum_fn():
    mesh = plsc.VectorSubcoreMesh(
        core_axis_name="c", subcore_axis_name="s",
        num_cores=_NC, num_subcores=_NS)

    @functools.partial(
        pl.kernel,
        out_type=jax.ShapeDtypeStruct((_B, _D), jnp.float32),
        mesh=mesh,
        scratch_types=[
            pltpu.VMEM((_IPW,), jnp.int32),
            pltpu.VMEM((_IPW, _D), jnp.float32),
            pltpu.VMEM((_BPW, _D), jnp.float32),
            pltpu.SemaphoreType.DMA,
        ],
        compiler_params=pltpu.CompilerParams(use_tc_tiling_on_sc=False),
    )
    def _gather_sum(idx_hbm, table_hbm, out_hbm, idx_v, rows_v, acc_v, sem):
        wid = lax.axis_index("s") * _NC + lax.axis_index("c")
        base = wid * _IPW
        pltpu.sync_copy(idx_hbm.at[pl.ds(base, _IPW)], idx_v)
        pltpu.async_copy(table_hbm.at[idx_v], rows_v, sem).wait()

        def row_body(b, carry):
            def ctx_body(c, accs):
                r = b * _CTX + c
                return tuple(accs[k] + rows_v[r, pl.ds(k * _L, _L)]
                             for k in range(_DCHUNKS))

            accs = lax.fori_loop(
                0, _CTX, ctx_body,
                tuple(jnp.zeros((_L,), jnp.float32) for _ in range(_DCHUNKS)))
            for k in range(_DCHUNKS):
                acc_v[b, pl.ds(k * _L, _L)] = accs[k]
            return carry

        lax.fori_loop(0, _BPW, row_body, 0)
        pltpu.sync_copy(acc_v, out_hbm.at[pl.ds(wid * _BPW, _BPW)])

    return _gather_sum


_BV = 1024
_NBLK = math.ceil(_V / _BV)


def _matmul_body(x_ref, w_ref, o_ref):
    x = x_ref[...].astype(jnp.bfloat16)
    w = w_ref[...].astype(jnp.bfloat16)
    o_ref[...] = lax.dot_general(
        x, w, (((1,), (1,)), ((), ())), preferred_element_type=jnp.float32)


def _scores(sum_embeds, out_embed):
    return pl.pallas_call(
        _matmul_body,
        grid=(_NBLK,),
        in_specs=[
            pl.BlockSpec((_B, _D), lambda i: (0, 0)),
            pl.BlockSpec((_BV, _D), lambda i: (i, 0)),
        ],
        out_specs=pl.BlockSpec((_B, _BV), lambda i: (0, i)),
        out_shape=jax.ShapeDtypeStruct((_B, _V), jnp.float32),
    )(sum_embeds, out_embed)


def kernel(context, in_embed, out_embed):
    sum_embeds = _gather_sum_fn()(context.reshape(-1), in_embed)
    return _scores(sum_embeds, out_embed)


# speedup vs baseline: 1.0400x; 1.0400x over previous
import functools
import math

import jax
import jax.numpy as jnp
from jax import lax
from jax.experimental import pallas as pl
from jax.experimental.pallas import tpu as pltpu
from jax.experimental.pallas import tpu_sc as plsc

_B, _CTX, _D, _V = 1024, 20, 64, 100000
_NC, _NS, _L = 2, 16, 16
_NW = _NC * _NS
_BPW = _B // _NW
_IPW = _BPW * _CTX
_DCHUNKS = _D // _L

@functools.cache
def _gather_sum_fn():
    mesh = plsc.VectorSubcoreMesh(
        core_axis_name="c", subcore_axis_name="s",
        num_cores=_NC, num_subcores=_NS)

    @functools.partial(
        pl.kernel,
        out_type=jax.ShapeDtypeStruct((_B, _D), jnp.float32),
        mesh=mesh,
        scratch_types=[
            pltpu.VMEM((_IPW,), jnp.int32),
            pltpu.VMEM((_IPW, _D), jnp.float32),
            pltpu.VMEM((_BPW, _D), jnp.float32),
            pltpu.SemaphoreType.DMA,
        ],
        compiler_params=pltpu.CompilerParams(use_tc_tiling_on_sc=False),
    )
    def _gather_sum(idx_hbm, table_hbm, out_hbm, idx_v, rows_v, acc_v, sem):
        wid = lax.axis_index("s") * _NC + lax.axis_index("c")
        base = wid * _IPW
        pltpu.sync_copy(idx_hbm.at[pl.ds(base, _IPW)], idx_v)
        pltpu.async_copy(table_hbm.at[idx_v], rows_v, sem).wait()

        def row_body(b, carry):
            def ctx_body(c, accs):
                r = b * _CTX + c
                return tuple(accs[k] + rows_v[r, pl.ds(k * _L, _L)]
                             for k in range(_DCHUNKS))

            accs = lax.fori_loop(
                0, _CTX, ctx_body,
                tuple(jnp.zeros((_L,), jnp.float32) for _ in range(_DCHUNKS)))
            for k in range(_DCHUNKS):
                acc_v[b, pl.ds(k * _L, _L)] = accs[k]
            return carry

        lax.fori_loop(0, _BPW, row_body, 0)
        pltpu.sync_copy(acc_v, out_hbm.at[pl.ds(wid * _BPW, _BPW)])

    return _gather_sum


_BV = 4096
_NBLK = math.ceil(_V / _BV)


def _matmul_body(x_ref, w_ref, o_ref):
    x = x_ref[...].astype(jnp.bfloat16)
    w = w_ref[...].astype(jnp.bfloat16)
    o_ref[...] = lax.dot_general(
        x, w, (((1,), (1,)), ((), ())), preferred_element_type=jnp.float32)


def _scores(sum_embeds, out_embed):
    return pl.pallas_call(
        _matmul_body,
        grid=(_NBLK,),
        in_specs=[
            pl.BlockSpec((_B, _D), lambda i: (0, 0)),
            pl.BlockSpec((_BV, _D), lambda i: (i, 0)),
        ],
        out_specs=pl.BlockSpec((_B, _BV), lambda i: (0, i)),
        out_shape=jax.ShapeDtypeStruct((_B, _V), jnp.float32),
    )(sum_embeds, out_embed)


def kernel(context, in_embed, out_embed):
    sum_embeds = _gather_sum_fn()(context.reshape(-1), in_embed)
    return _scores(sum_embeds, out_embed)


# speedup vs baseline: 1.1056x; 1.0631x over previous
import functools
import math

import jax
import jax.numpy as jnp
from jax import lax
from jax.experimental import pallas as pl
from jax.experimental.pallas import tpu as pltpu
from jax.experimental.pallas import tpu_sc as plsc

_B, _CTX, _D, _V = 1024, 20, 64, 100000
_NC, _NS, _L = 2, 16, 16
_NW = _NC * _NS
_BPW = _B // _NW
_IPW = _BPW * _CTX
_DCHUNKS = _D // _L

@functools.cache
def _gather_sum_fn():
    mesh = plsc.VectorSubcoreMesh(
        core_axis_name="c", subcore_axis_name="s",
        num_cores=_NC, num_subcores=_NS)

    @functools.partial(
        pl.kernel,
        out_type=jax.ShapeDtypeStruct((_B, _D), jnp.float32),
        mesh=mesh,
        scratch_types=[
            pltpu.VMEM((_IPW,), jnp.int32),
            pltpu.VMEM((_IPW, _D), jnp.float32),
            pltpu.VMEM((_BPW, _D), jnp.float32),
            pltpu.SemaphoreType.DMA,
        ],
        compiler_params=pltpu.CompilerParams(use_tc_tiling_on_sc=False),
    )
    def _gather_sum(idx_hbm, table_hbm, out_hbm, idx_v, rows_v, acc_v, sem):
        wid = lax.axis_index("s") * _NC + lax.axis_index("c")
        base = wid * _IPW
        pltpu.sync_copy(idx_hbm.at[pl.ds(base, _IPW)], idx_v)
        pltpu.async_copy(table_hbm.at[idx_v], rows_v, sem).wait()

        def row_body(b, carry):
            def ctx_body(c, accs):
                r = b * _CTX + c
                return tuple(accs[k] + rows_v[r, pl.ds(k * _L, _L)]
                             for k in range(_DCHUNKS))

            accs = lax.fori_loop(
                0, _CTX, ctx_body,
                tuple(jnp.zeros((_L,), jnp.float32) for _ in range(_DCHUNKS)))
            for k in range(_DCHUNKS):
                acc_v[b, pl.ds(k * _L, _L)] = accs[k]
            return carry

        lax.fori_loop(0, _BPW, row_body, 0)
        pltpu.sync_copy(acc_v, out_hbm.at[pl.ds(wid * _BPW, _BPW)])

    return _gather_sum


_BM = 32
_NBLK = _B // _BM


def _matmul_body(x_ref, wt_ref, o_ref):
    x = x_ref[...].astype(jnp.bfloat16)
    o_ref[...] = lax.dot_general(
        x, wt_ref[...], (((1,), (0,)), ((), ())),
        preferred_element_type=jnp.float32)


def _scores(sum_embeds, out_embed_t):
    return pl.pallas_call(
        _matmul_body,
        grid=(_NBLK,),
        in_specs=[
            pl.BlockSpec((_BM, _D), lambda i: (i, 0)),
            pl.BlockSpec((_D, _V), lambda i: (0, 0)),
        ],
        out_specs=pl.BlockSpec((_BM, _V), lambda i: (i, 0)),
        out_shape=jax.ShapeDtypeStruct((_B, _V), jnp.float32),
    )(sum_embeds, out_embed_t)


def kernel(context, in_embed, out_embed):
    sum_embeds = _gather_sum_fn()(context.reshape(-1), in_embed)
    return _scores(sum_embeds, out_embed.T.astype(jnp.bfloat16))


# speedup vs baseline: 1.1108x; 1.0047x over previous
import functools
import math

import jax
import jax.numpy as jnp
from jax import lax
from jax.experimental import pallas as pl
from jax.experimental.pallas import tpu as pltpu
from jax.experimental.pallas import tpu_sc as plsc

_B, _CTX, _D, _V = 1024, 20, 64, 100000
_NC, _NS, _L = 2, 16, 16
_NW = _NC * _NS
_BPW = _B // _NW
_IPW = _BPW * _CTX
_DCHUNKS = _D // _L

@functools.cache
def _gather_sum_fn():
    mesh = plsc.VectorSubcoreMesh(
        core_axis_name="c", subcore_axis_name="s",
        num_cores=_NC, num_subcores=_NS)

    @functools.partial(
        pl.kernel,
        out_type=jax.ShapeDtypeStruct((_B, _D), jnp.float32),
        mesh=mesh,
        scratch_types=[
            pltpu.VMEM((_IPW,), jnp.int32),
            pltpu.VMEM((_IPW, _D), jnp.float32),
            pltpu.VMEM((_BPW, _D), jnp.float32),
            pltpu.SemaphoreType.DMA,
        ],
        compiler_params=pltpu.CompilerParams(use_tc_tiling_on_sc=False),
    )
    def _gather_sum(idx_hbm, table_hbm, out_hbm, idx_v, rows_v, acc_v, sem):
        wid = lax.axis_index("s") * _NC + lax.axis_index("c")
        base = wid * _IPW
        pltpu.sync_copy(idx_hbm.at[pl.ds(base, _IPW)], idx_v)
        pltpu.async_copy(table_hbm.at[idx_v], rows_v, sem).wait()

        def row_body(b, carry):
            def ctx_body(c, accs):
                r = b * _CTX + c
                return tuple(accs[k] + rows_v[r, pl.ds(k * _L, _L)]
                             for k in range(_DCHUNKS))

            accs = lax.fori_loop(
                0, _CTX, ctx_body,
                tuple(jnp.zeros((_L,), jnp.float32) for _ in range(_DCHUNKS)))
            for k in range(_DCHUNKS):
                acc_v[b, pl.ds(k * _L, _L)] = accs[k]
            return carry

        lax.fori_loop(0, _BPW, row_body, 0)
        pltpu.sync_copy(acc_v, out_hbm.at[pl.ds(wid * _BPW, _BPW)])

    return _gather_sum


_BM = 16
_NBLK = _B // _BM
_NBUF = 4


def _matmul_body(x_ref, wt_ref, o_hbm, bufs, sems):
    i = pl.program_id(0)
    slot = lax.rem(i, _NBUF)

    @pl.when(i >= _NBUF)
    def _drain_slot():
        for k in range(_NBUF):
            @pl.when(slot == k)
            def _():
                pltpu.make_async_copy(
                    bufs.at[k],
                    o_hbm.at[pl.ds((i - _NBUF) * _BM, _BM)],
                    sems.at[k]).wait()

    x = x_ref[...].astype(jnp.bfloat16)
    bufs[slot] = lax.dot_general(
        x, wt_ref[...], (((1,), (0,)), ((), ())),
        preferred_element_type=jnp.float32)

    for k in range(_NBUF):
        @pl.when(slot == k)
        def _start_slot():
            pltpu.make_async_copy(
                bufs.at[k], o_hbm.at[pl.ds(i * _BM, _BM)], sems.at[k]).start()

    @pl.when(i == _NBLK - 1)
    def _final_drain():
        for j in range(_NBUF):
            jj = _NBLK - _NBUF + j
            k = jj % _NBUF
            pltpu.make_async_copy(
                bufs.at[k], o_hbm.at[pl.ds(jj * _BM, _BM)], sems.at[k]).wait()


def _scores(sum_embeds, out_embed_t):
    return pl.pallas_call(
        _matmul_body,
        grid=(_NBLK,),
        in_specs=[
            pl.BlockSpec((_BM, _D), lambda i: (i, 0)),
            pl.BlockSpec((_D, _V), lambda i: (0, 0)),
        ],
        out_specs=pl.BlockSpec(memory_space=pltpu.HBM),
        out_shape=jax.ShapeDtypeStruct((_B, _V), jnp.float32),
        scratch_shapes=[
            pltpu.VMEM((_NBUF, _BM, _V), jnp.float32),
            pltpu.SemaphoreType.DMA((_NBUF,)),
        ],
    )(sum_embeds, out_embed_t)


def kernel(context, in_embed, out_embed):
    sum_embeds = _gather_sum_fn()(context.reshape(-1), in_embed)
    return _scores(sum_embeds, out_embed.T.astype(jnp.bfloat16))
